# baseline (device time: 37578 ns/iter reference)
import jax
import jax.numpy as jnp
from jax import lax
from jax.experimental import pallas as pl
from jax.experimental.pallas import tpu as pltpu

N_DEV = 8
N_LAYERS = 3


def kernel(x, Win0, Wout0, Win1, Wout1, Win2, Wout2):
    b, d_shard = x.shape
    h_dim = Win0.shape[1]
    blk = h_dim // N_DEV

    def body(x_ref, win0_ref, wout0_ref, win1_ref, wout1_ref, win2_ref,
             wout2_ref, out_ref, part_ref, rs_ref, ag_ref,
             send_sems, recv_sems):
        my_pos = lax.axis_index("i")

        barrier_sem = pltpu.get_barrier_semaphore()
        for d in range(1, N_DEV):
            pl.semaphore_signal(
                barrier_sem, inc=1,
                device_id=((my_pos + d) % N_DEV,),
                device_id_type=pl.DeviceIdType.MESH,
            )
        pl.semaphore_wait(barrier_sem, N_DEV - 1)

        wins = [win0_ref, win1_ref, win2_ref]
        wouts = [wout0_ref, wout1_ref, wout2_ref]

        x_cur = x_ref[:, :]
        for l in range(N_LAYERS):
            partial = jnp.dot(x_cur, wins[l][:, :],
                              preferred_element_type=jnp.float32)
            for t in range(N_DEV):
                part_ref[l, t, :, :] = (
                    partial[:, t * blk:(t + 1) * blk].astype(jnp.bfloat16))

            rs_rdmas = []
            for d in range(1, N_DEV):
                t = (my_pos + d) % N_DEV
                rdma = pltpu.make_async_remote_copy(
                    src_ref=part_ref.at[l, t],
                    dst_ref=rs_ref.at[l, d - 1],
                    send_sem=send_sems.at[l, 0, d - 1],
                    recv_sem=recv_sems.at[l, 0, d - 1],
                    device_id=(t,),
                    device_id_type=pl.DeviceIdType.MESH,
                )
                rdma.start()
                rs_rdmas.append(rdma)

            acc = part_ref[l, my_pos, :, :].astype(jnp.float32)
            for d in range(1, N_DEV):
                rs_rdmas[d - 1].wait_recv()
                acc = acc + rs_ref[l, d - 1, :, :].astype(jnp.float32)
            hred = jnp.maximum(acc, 0.0).astype(jnp.bfloat16)
            ag_ref[l, my_pos, :, :] = hred

            ag_rdmas = []
            for d in range(1, N_DEV):
                t = (my_pos + d) % N_DEV
                rdma = pltpu.make_async_remote_copy(
                    src_ref=ag_ref.at[l, my_pos],
                    dst_ref=ag_ref.at[l, my_pos],
                    send_sem=send_sems.at[l, 1, d - 1],
                    recv_sem=recv_sems.at[l, 1, d - 1],
                    device_id=(t,),
                    device_id_type=pl.DeviceIdType.MESH,
                )
                rdma.start()
                ag_rdmas.append(rdma)
            for rdma in ag_rdmas:
                rdma.wait_recv()

            nxt = jnp.dot(ag_ref[l, 0, :, :].astype(jnp.float32),
                          wouts[l][0:blk, :],
                          preferred_element_type=jnp.float32)
            for t in range(1, N_DEV):
                nxt = nxt + jnp.dot(
                    ag_ref[l, t, :, :].astype(jnp.float32),
                    wouts[l][t * blk:(t + 1) * blk, :],
                    preferred_element_type=jnp.float32)
            if l == N_LAYERS - 1:
                out_ref[:, :] = nxt
            else:
                x_cur = nxt

            for rdma in rs_rdmas:
                rdma.wait_send()
            for rdma in ag_rdmas:
                rdma.wait_send()

    return pl.pallas_call(
        body,
        out_shape=jax.ShapeDtypeStruct((b, d_shard), jnp.float32),
        in_specs=[pl.BlockSpec(memory_space=pltpu.VMEM)] * 7,
        out_specs=pl.BlockSpec(memory_space=pltpu.VMEM),
        scratch_shapes=[
            pltpu.VMEM((N_LAYERS, N_DEV, b, blk), jnp.bfloat16),
            pltpu.VMEM((N_LAYERS, N_DEV - 1, b, blk), jnp.bfloat16),
            pltpu.VMEM((N_LAYERS, N_DEV, b, blk), jnp.bfloat16),
            pltpu.SemaphoreType.DMA((N_LAYERS, 2, N_DEV - 1)),
            pltpu.SemaphoreType.DMA((N_LAYERS, 2, N_DEV - 1)),
        ],
        compiler_params=pltpu.CompilerParams(collective_id=0),
    )(x, Win0, Wout0, Win1, Wout1, Win2, Wout2)


# device time: 37499 ns/iter; 1.0021x vs baseline; 1.0021x over previous
import jax
import jax.numpy as jnp
from jax import lax
from jax.experimental import pallas as pl
from jax.experimental.pallas import tpu as pltpu

N_DEV = 8
N_LAYERS = 3


def kernel(x, Win0, Wout0, Win1, Wout1, Win2, Wout2):
    b, d_shard = x.shape
    h_dim = Win0.shape[1]
    blk = h_dim // N_DEV

    def body(x_ref, win0_ref, wout0_ref, win1_ref, wout1_ref, win2_ref,
             wout2_ref, out_ref, part_ref, rs_ref, ag_ref,
             wbf_in, wbf_out, send_sems, recv_sems):
        my_pos = lax.axis_index("i")

        barrier_sem = pltpu.get_barrier_semaphore()
        for d in range(1, N_DEV):
            pl.semaphore_signal(
                barrier_sem, inc=1,
                device_id=((my_pos + d) % N_DEV,),
                device_id_type=pl.DeviceIdType.MESH,
            )
        pl.semaphore_wait(barrier_sem, N_DEV - 1)

        wins = [win0_ref, win1_ref, win2_ref]
        wouts = [wout0_ref, wout1_ref, wout2_ref]

        wbf_in[0, :, :] = wins[0][:, :].astype(jnp.bfloat16)
        x_cur = x_ref[:, :].astype(jnp.bfloat16)

        for l in range(N_LAYERS):
            partial = jnp.dot(x_cur, wbf_in[l, :, :],
                              preferred_element_type=jnp.float32)
            for t in range(N_DEV):
                part_ref[l, t, :, :] = (
                    partial[:, t * blk:(t + 1) * blk].astype(jnp.bfloat16))

            rs_rdmas = []
            for d in range(1, N_DEV):
                t = (my_pos + d) % N_DEV
                rdma = pltpu.make_async_remote_copy(
                    src_ref=part_ref.at[l, t],
                    dst_ref=rs_ref.at[l, d - 1],
                    send_sem=send_sems.at[l, 0, d - 1],
                    recv_sem=recv_sems.at[l, 0, d - 1],
                    device_id=(t,),
                    device_id_type=pl.DeviceIdType.MESH,
                )
                rdma.start()
                rs_rdmas.append(rdma)

            wbf_out[l, :, :] = wouts[l][:, :].astype(jnp.bfloat16)
            if l + 1 < N_LAYERS:
                wbf_in[l + 1, :, :] = wins[l + 1][:, :].astype(jnp.bfloat16)

            acc = part_ref[l, my_pos, :, :].astype(jnp.float32)
            for d in range(1, N_DEV):
                rs_rdmas[d - 1].wait_recv()
                acc = acc + rs_ref[l, d - 1, :, :].astype(jnp.float32)
            hred = jnp.maximum(acc, 0.0).astype(jnp.bfloat16)
            ag_ref[l, my_pos, :, :] = hred

            ag_rdmas = []
            for d in range(1, N_DEV):
                t = (my_pos + d) % N_DEV
                rdma = pltpu.make_async_remote_copy(
                    src_ref=ag_ref.at[l, my_pos],
                    dst_ref=ag_ref.at[l, my_pos],
                    send_sem=send_sems.at[l, 1, d - 1],
                    recv_sem=recv_sems.at[l, 1, d - 1],
                    device_id=(t,),
                    device_id_type=pl.DeviceIdType.MESH,
                )
                rdma.start()
                ag_rdmas.append(rdma)
            for rdma in ag_rdmas:
                rdma.wait_recv()

            nxt = jnp.dot(ag_ref[l, 0, :, :], wbf_out[l, 0:blk, :],
                          preferred_element_type=jnp.float32)
            for t in range(1, N_DEV):
                nxt = nxt + jnp.dot(
                    ag_ref[l, t, :, :],
                    wbf_out[l, t * blk:(t + 1) * blk, :],
                    preferred_element_type=jnp.float32)
            if l == N_LAYERS - 1:
                out_ref[:, :] = nxt
            else:
                x_cur = nxt.astype(jnp.bfloat16)

            for rdma in rs_rdmas:
                rdma.wait_send()
            for rdma in ag_rdmas:
                rdma.wait_send()

    return pl.pallas_call(
        body,
        out_shape=jax.ShapeDtypeStruct((b, d_shard), jnp.float32),
        in_specs=[pl.BlockSpec(memory_space=pltpu.VMEM)] * 7,
        out_specs=pl.BlockSpec(memory_space=pltpu.VMEM),
        scratch_shapes=[
            pltpu.VMEM((N_LAYERS, N_DEV, b, blk), jnp.bfloat16),
            pltpu.VMEM((N_LAYERS, N_DEV - 1, b, blk), jnp.bfloat16),
            pltpu.VMEM((N_LAYERS, N_DEV, b, blk), jnp.bfloat16),
            pltpu.VMEM((N_LAYERS, Win0.shape[0], h_dim), jnp.bfloat16),
            pltpu.VMEM((N_LAYERS, h_dim, d_shard), jnp.bfloat16),
            pltpu.SemaphoreType.DMA((N_LAYERS, 2, N_DEV - 1)),
            pltpu.SemaphoreType.DMA((N_LAYERS, 2, N_DEV - 1)),
        ],
        compiler_params=pltpu.CompilerParams(collective_id=0),
    )(x, Win0, Wout0, Win1, Wout1, Win2, Wout2)
